# baseline (speedup 1.0000x reference)
import functools

import jax
import jax.numpy as jnp
from jax import lax
from jax.experimental import pallas as pl
from jax.experimental.pallas import tpu as pltpu
from jax.experimental.pallas import tpu_sc as plsc

TOP_K = 8
NUM_EXPERTS = 64
NUM_TOKENS = 16384
BLOCK_T = 1024
NUM_WORKERS = 32
TPW = NUM_TOKENS // NUM_WORKERS


def _matmul_body(x_ref, w_ref, logits_ref):
    logits_ref[...] = jax.lax.dot_general(
        x_ref[...], w_ref[...], (((1,), (1,)), ((), ())),
        preferred_element_type=jnp.float32,
    )


def _tc_logits(hidden_states, W):
    nt, hd = hidden_states.shape
    ne = W.shape[0]
    return pl.pallas_call(
        _matmul_body,
        grid=(nt // BLOCK_T,),
        in_specs=[
            pl.BlockSpec((BLOCK_T, hd), lambda i: (i, 0)),
            pl.BlockSpec((ne, hd), lambda i: (0, 0)),
        ],
        out_specs=pl.BlockSpec((BLOCK_T, ne), lambda i: (i, 0)),
        out_shape=jax.ShapeDtypeStruct((nt, ne), jnp.float32),
    )(hidden_states, W)


def _merge_desc(ka, va, kb, vb):
    rk = lax.rev(kb, (0,))
    rv = lax.rev(vb, (0,))
    take_a = ka >= rk
    hk = jnp.where(take_a, ka, rk)
    hv = jnp.where(take_a, va, rv)
    return plsc.sort_key_val(hk, hv, descending=True)


def _sc_topk_body(logits_hbm, w_hbm, e_hbm, slab, wout, eout, sem):
    wid = lax.axis_index("s") * 2 + lax.axis_index("c")
    base = wid * TPW
    pltpu.sync_copy(logits_hbm.at[pl.ds(base * NUM_EXPERTS, TPW * NUM_EXPERTS)], slab)
    lane = lax.iota(jnp.int32, 16)
    first8 = lane < TOP_K

    def body(t, carry):
        ks, vs = [], []
        for c in range(4):
            k, v = plsc.sort_key_val(
                slab[pl.ds(t * NUM_EXPERTS + c * 16, 16)],
                lane + c * 16,
                descending=True,
            )
            ks.append(k)
            vs.append(v)
        k01, v01 = _merge_desc(ks[0], vs[0], ks[1], vs[1])
        k23, v23 = _merge_desc(ks[2], vs[2], ks[3], vs[3])
        kt, vt = _merge_desc(k01, v01, k23, v23)
        mx = jnp.max(kt)
        ex = jnp.where(first8, jnp.exp(kt - mx), 0.0)
        w = ex / jnp.sum(ex)
        wout[pl.ds(t * 16, 16)] = w
        eout[pl.ds(t * 16, 16)] = vt
        return carry

    lax.fori_loop(0, TPW, body, 0)
    pltpu.sync_copy(wout, w_hbm.at[pl.ds(base * 16, TPW * 16)])
    pltpu.sync_copy(eout, e_hbm.at[pl.ds(base * 16, TPW * 16)])


_sc_topk = functools.partial(
    pl.kernel,
    mesh=plsc.VectorSubcoreMesh(core_axis_name="c", subcore_axis_name="s"),
    compiler_params=pltpu.CompilerParams(needs_layout_passes=False),
    out_type=[
        jax.ShapeDtypeStruct((NUM_TOKENS * 16,), jnp.float32),
        jax.ShapeDtypeStruct((NUM_TOKENS * 16,), jnp.int32),
    ],
    scratch_types=[
        pltpu.VMEM((TPW * NUM_EXPERTS,), jnp.float32),
        pltpu.VMEM((TPW * 16,), jnp.float32),
        pltpu.VMEM((TPW * 16,), jnp.int32),
        pltpu.SemaphoreType.DMA,
    ],
)(_sc_topk_body)


def kernel(hidden_states, W):
    logits = _tc_logits(hidden_states, W)
    wflat, eflat = _sc_topk(logits.reshape(-1))
    weights = wflat.reshape(NUM_TOKENS, 16)[:, :TOP_K]
    experts = eflat.reshape(NUM_TOKENS, 16)[:, :TOP_K]
    return (weights, experts, logits)


# speedup vs baseline: 1.0001x; 1.0001x over previous
import functools

import jax
import jax.numpy as jnp
from jax import lax
from jax.experimental import pallas as pl
from jax.experimental.pallas import tpu as pltpu
from jax.experimental.pallas import tpu_sc as plsc

TOP_K = 8
NUM_EXPERTS = 64
NUM_TOKENS = 16384
BLOCK_T = 1024
NUM_WORKERS = 32
TPW = NUM_TOKENS // NUM_WORKERS


def _matmul_body(x_ref, w_ref, logits_ref):
    logits_ref[...] = jax.lax.dot_general(
        x_ref[...], w_ref[...], (((1,), (1,)), ((), ())),
        preferred_element_type=jnp.float32,
    )


def _tc_logits(hidden_states, W):
    nt, hd = hidden_states.shape
    ne = W.shape[0]
    return pl.pallas_call(
        _matmul_body,
        grid=(nt // BLOCK_T,),
        in_specs=[
            pl.BlockSpec((BLOCK_T, hd), lambda i: (i, 0)),
            pl.BlockSpec((ne, hd), lambda i: (0, 0)),
        ],
        out_specs=pl.BlockSpec((BLOCK_T, ne), lambda i: (i, 0)),
        out_shape=jax.ShapeDtypeStruct((nt, ne), jnp.float32),
    )(hidden_states, W)


def _merge_desc(ka, va, kb, vb):
    rk = lax.rev(kb, (0,))
    rv = lax.rev(vb, (0,))
    take_a = ka >= rk
    hk = jnp.where(take_a, ka, rk)
    hv = jnp.where(take_a, va, rv)
    return plsc.sort_key_val(hk, hv, descending=True)


def _sc_topk_body(logits_hbm, w_hbm, e_hbm, slab, wout, eout, sem):
    wid = lax.axis_index("s") * 2 + lax.axis_index("c")
    base = wid * TPW
    pltpu.sync_copy(logits_hbm.at[pl.ds(base * NUM_EXPERTS, TPW * NUM_EXPERTS)], slab)
    lane = lax.iota(jnp.int32, 16)
    first8 = lane < TOP_K

    def body(tq, carry):
        for u in range(4):
            t = tq * 4 + u
            ks, vs = [], []
            for c in range(4):
                k, v = plsc.sort_key_val(
                    slab[pl.ds(t * NUM_EXPERTS + c * 16, 16)],
                    lane + c * 16,
                    descending=True,
                )
                ks.append(k)
                vs.append(v)
            k01, v01 = _merge_desc(ks[0], vs[0], ks[1], vs[1])
            k23, v23 = _merge_desc(ks[2], vs[2], ks[3], vs[3])
            kt, vt = _merge_desc(k01, v01, k23, v23)
            mx = jnp.max(kt)
            ex = jnp.where(first8, jnp.exp(kt - mx), 0.0)
            w = ex / jnp.sum(ex)
            wout[pl.ds(t * 16, 16)] = w
            eout[pl.ds(t * 16, 16)] = vt
        return carry

    lax.fori_loop(0, TPW // 4, body, 0)
    pltpu.sync_copy(wout, w_hbm.at[pl.ds(base * 16, TPW * 16)])
    pltpu.sync_copy(eout, e_hbm.at[pl.ds(base * 16, TPW * 16)])


_sc_topk = functools.partial(
    pl.kernel,
    mesh=plsc.VectorSubcoreMesh(core_axis_name="c", subcore_axis_name="s"),
    compiler_params=pltpu.CompilerParams(needs_layout_passes=False),
    out_type=[
        jax.ShapeDtypeStruct((NUM_TOKENS * 16,), jnp.float32),
        jax.ShapeDtypeStruct((NUM_TOKENS * 16,), jnp.int32),
    ],
    scratch_types=[
        pltpu.VMEM((TPW * NUM_EXPERTS,), jnp.float32),
        pltpu.VMEM((TPW * 16,), jnp.float32),
        pltpu.VMEM((TPW * 16,), jnp.int32),
        pltpu.SemaphoreType.DMA,
    ],
)(_sc_topk_body)


def kernel(hidden_states, W):
    logits = _tc_logits(hidden_states, W)
    wflat, eflat = _sc_topk(logits.reshape(-1))
    weights = wflat.reshape(NUM_TOKENS, 16)[:, :TOP_K]
    experts = eflat.reshape(NUM_TOKENS, 16)[:, :TOP_K]
    return (weights, experts, logits)


# speedup vs baseline: 1.2523x; 1.2523x over previous
import functools

import jax
import jax.numpy as jnp
from jax import lax
from jax.experimental import pallas as pl
from jax.experimental.pallas import tpu as pltpu
from jax.experimental.pallas import tpu_sc as plsc

TOP_K = 8
NUM_EXPERTS = 64
NUM_TOKENS = 16384
BLOCK_T = 1024
NUM_WORKERS = 32
TPW = NUM_TOKENS // NUM_WORKERS


def _matmul_body(x_ref, w_ref, logits_ref):
    logits_ref[...] = jax.lax.dot_general(
        x_ref[...], w_ref[...], (((1,), (1,)), ((), ())),
        preferred_element_type=jnp.float32,
    )


def _tc_logits(hidden_states, W):
    nt, hd = hidden_states.shape
    ne = W.shape[0]
    return pl.pallas_call(
        _matmul_body,
        grid=(nt // BLOCK_T,),
        in_specs=[
            pl.BlockSpec((BLOCK_T, hd), lambda i: (i, 0)),
            pl.BlockSpec((ne, hd), lambda i: (0, 0)),
        ],
        out_specs=pl.BlockSpec((BLOCK_T, ne), lambda i: (i, 0)),
        out_shape=jax.ShapeDtypeStruct((nt, ne), jnp.float32),
    )(hidden_states, W)


def _merge_desc(ka, va, kb, vb):
    rk = lax.rev(kb, (0,))
    rv = lax.rev(vb, (0,))
    take_a = ka >= rk
    hk = jnp.where(take_a, ka, rk)
    hv = jnp.where(take_a, va, rv)
    return plsc.sort_key_val(hk, hv, descending=True)


def _sc_topk_body(logits_hbm, w_hbm, e_hbm, slab, wout, eout, sem):
    wid = lax.axis_index("s") * 2 + lax.axis_index("c")
    base = wid * TPW
    pltpu.sync_copy(logits_hbm.at[pl.ds(base * NUM_EXPERTS, TPW * NUM_EXPERTS)], slab)
    lane = lax.iota(jnp.int32, 16)
    first8 = lane < TOP_K

    @plsc.parallel_loop(0, TPW, 4, unroll=2)
    def body(tbase):
        for u in range(4):
            t = tbase + u
            ks, vs = [], []
            for c in range(4):
                k, v = plsc.sort_key_val(
                    slab[pl.ds(t * NUM_EXPERTS + c * 16, 16)],
                    lane + c * 16,
                    descending=True,
                )
                ks.append(k)
                vs.append(v)
            k01, v01 = _merge_desc(ks[0], vs[0], ks[1], vs[1])
            k23, v23 = _merge_desc(ks[2], vs[2], ks[3], vs[3])
            kt, vt = _merge_desc(k01, v01, k23, v23)
            mx = jnp.max(kt)
            ex = jnp.where(first8, jnp.exp(kt - mx), 0.0)
            w = ex / jnp.sum(ex)
            wout[pl.ds(t * 16, 16)] = w
            eout[pl.ds(t * 16, 16)] = vt
    pltpu.sync_copy(wout, w_hbm.at[pl.ds(base * 16, TPW * 16)])
    pltpu.sync_copy(eout, e_hbm.at[pl.ds(base * 16, TPW * 16)])


_sc_topk = functools.partial(
    pl.kernel,
    mesh=plsc.VectorSubcoreMesh(core_axis_name="c", subcore_axis_name="s"),
    compiler_params=pltpu.CompilerParams(needs_layout_passes=False),
    out_type=[
        jax.ShapeDtypeStruct((NUM_TOKENS * 16,), jnp.float32),
        jax.ShapeDtypeStruct((NUM_TOKENS * 16,), jnp.int32),
    ],
    scratch_types=[
        pltpu.VMEM((TPW * NUM_EXPERTS,), jnp.float32),
        pltpu.VMEM((TPW * 16,), jnp.float32),
        pltpu.VMEM((TPW * 16,), jnp.int32),
        pltpu.SemaphoreType.DMA,
    ],
)(_sc_topk_body)


def kernel(hidden_states, W):
    logits = _tc_logits(hidden_states, W)
    wflat, eflat = _sc_topk(logits.reshape(-1))
    weights = wflat.reshape(NUM_TOKENS, 16)[:, :TOP_K]
    experts = eflat.reshape(NUM_TOKENS, 16)[:, :TOP_K]
    return (weights, experts, logits)
